# Initial kernel scaffold; baseline (speedup 1.0000x reference)
import jax
import jax.numpy as jnp
from jax import lax
from jax.experimental import pallas as pl
from jax.experimental.pallas import tpu as pltpu
from jax.experimental.pallas import tpu_sc as plsc

B, C, N_BIN = 256, 128, 360
MAX_SHIFT = 15
LOWER_BIN = 15
N_OUT = N_BIN - 2 * MAX_SHIFT

NUM_WORKERS = 32
B_PER_W = B // NUM_WORKERS


def _sc_body(spec_hbm, ns_hbm, x_hbm, xs_hbm, ns_v, buf0, buf1, sem0, sem1):
    wid = lax.axis_index("s") * 2 + lax.axis_index("c")
    base = wid * B_PER_W

    pltpu.sync_copy(ns_hbm.at[pl.ds(base, B_PER_W)], ns_v)

    bufs = (buf0, buf1)
    sems = (sem0, sem1)
    cps = [None, None]
    cps[0] = pltpu.async_copy(spec_hbm.at[base], buf0, sem0)
    for i in range(B_PER_W):
        b = base + i
        if i + 1 < B_PER_W:
            cps[(i + 1) % 2] = pltpu.async_copy(
                spec_hbm.at[b + 1], bufs[(i + 1) % 2], sems[(i + 1) % 2]
            )
        buf = bufs[i % 2]
        cps[i % 2].wait()
        fb = LOWER_BIN - ns_v[i]
        pltpu.sync_copy(buf.at[:, pl.ds(LOWER_BIN, N_OUT)], x_hbm.at[b])
        pltpu.sync_copy(buf.at[:, pl.ds(fb, N_OUT)], xs_hbm.at[b])


def kernel(spec, n_shifts):
    ns32 = n_shifts.astype(jnp.int32)
    mesh = plsc.VectorSubcoreMesh(core_axis_name="c", subcore_axis_name="s")
    x, xs = pl.kernel(
        _sc_body,
        out_type=(
            jax.ShapeDtypeStruct((B, C, N_OUT), jnp.float32),
            jax.ShapeDtypeStruct((B, C, N_OUT), jnp.float32),
        ),
        mesh=mesh,
        scratch_types=(
            pltpu.VMEM((B_PER_W,), jnp.int32),
            pltpu.VMEM((C, N_BIN), jnp.float32),
            pltpu.VMEM((C, N_BIN), jnp.float32),
            pltpu.SemaphoreType.DMA,
            pltpu.SemaphoreType.DMA,
        ),
    )(spec, ns32)
    return (x, xs, n_shifts)


# speedup vs baseline: 1.3519x; 1.3519x over previous
import jax
import jax.numpy as jnp
from jax import lax
from jax.experimental import pallas as pl
from jax.experimental.pallas import tpu as pltpu
from jax.experimental.pallas import tpu_sc as plsc

B, C, N_BIN = 256, 128, 360
MAX_SHIFT = 15
LOWER_BIN = 15
N_OUT = N_BIN - 2 * MAX_SHIFT

NUM_WORKERS = 32
B_PER_W = B // NUM_WORKERS

IN_W = C * N_BIN
OUT_W = C * N_OUT
PAD = 8
K16 = (N_OUT + 15) // 16


def _sc_body(spec_hbm, ns_hbm, x_hbm, xs_hbm, ns_v, inb, xb, xsb, sem):
    wid = lax.axis_index("s") * 2 + lax.axis_index("c")
    base = wid * B_PER_W

    pltpu.sync_copy(ns_hbm.at[pl.ds(base, B_PER_W)], ns_v.at[pl.ds(0, B_PER_W)])
    ns_vec = ns_v[...]

    for i in range(B_PER_W):
        b = base + i
        pltpu.sync_copy(spec_hbm.at[b], inb.at[pl.ds(0, IN_W)])
        fb = LOWER_BIN - ns_vec[i]

        def row_body(r, _):
            src = r * N_BIN
            dst = r * N_OUT
            for k in range(K16):
                xb[pl.ds(dst + 16 * k, 16)] = inb[pl.ds(src + LOWER_BIN + 16 * k, 16)]
                xsb[pl.ds(dst + 16 * k, 16)] = inb[pl.ds(src + fb + 16 * k, 16)]
            return _

        lax.fori_loop(0, C, row_body, None)
        pltpu.sync_copy(xb.at[pl.ds(0, OUT_W)], x_hbm.at[b])
        pltpu.sync_copy(xsb.at[pl.ds(0, OUT_W)], xs_hbm.at[b])


def kernel(spec, n_shifts):
    ns32 = n_shifts.astype(jnp.int32)
    spec_flat = spec.reshape(B, IN_W)
    mesh = plsc.VectorSubcoreMesh(core_axis_name="c", subcore_axis_name="s")
    x, xs = pl.kernel(
        _sc_body,
        out_type=(
            jax.ShapeDtypeStruct((B, OUT_W), jnp.float32),
            jax.ShapeDtypeStruct((B, OUT_W), jnp.float32),
        ),
        mesh=mesh,
        compiler_params=pltpu.CompilerParams(use_tc_tiling_on_sc=False),
        scratch_types=(
            pltpu.VMEM((16,), jnp.int32),
            pltpu.VMEM((IN_W + PAD,), jnp.float32),
            pltpu.VMEM((OUT_W + PAD,), jnp.float32),
            pltpu.VMEM((OUT_W + PAD,), jnp.float32),
            pltpu.SemaphoreType.DMA,
        ),
    )(spec_flat, ns32)
    return (x.reshape(B, C, N_OUT), xs.reshape(B, C, N_OUT), n_shifts)


# speedup vs baseline: 1.5452x; 1.1430x over previous
import jax
import jax.numpy as jnp
from jax import lax
from jax.experimental import pallas as pl
from jax.experimental.pallas import tpu as pltpu
from jax.experimental.pallas import tpu_sc as plsc

B, C, N_BIN = 256, 128, 360
MAX_SHIFT = 15
LOWER_BIN = 15
N_OUT = N_BIN - 2 * MAX_SHIFT

NUM_WORKERS = 32
B_PER_W = B // NUM_WORKERS
R_CHUNK = 32
CH_PER_B = C // R_CHUNK
K_FULL = N_OUT // 16
TAIL = N_OUT - 16


def _sc_body(spec_hbm, ns_hbm, x_hbm, xs_hbm, ns_v,
             inb0, inb1, xb0, xb1, xsb0, xsb1,
             isem0, isem1, xsem0, xsem1, ssem0, ssem1):
    wid = lax.axis_index("s") * 2 + lax.axis_index("c")
    base = wid * B_PER_W

    pltpu.sync_copy(ns_hbm.at[pl.ds(base, B_PER_W)], ns_v.at[pl.ds(0, B_PER_W)])
    ns_vec = ns_v[...]
    fbs = [LOWER_BIN - ns_vec[j] for j in range(B_PER_W)]

    inb = (inb0, inb1)
    xb = (xb0, xb1)
    xsb = (xsb0, xsb1)
    isem = (isem0, isem1)
    xsem = (xsem0, xsem1)
    ssem = (ssem0, ssem1)

    def in_copy(i, c, p):
        return pltpu.make_async_copy(
            spec_hbm.at[base + i, pl.ds(c * R_CHUNK, R_CHUNK), :], inb[p], isem[p]
        )

    def out_copies(i, c, p):
        rows = pl.ds(c * R_CHUNK, R_CHUNK)
        return (
            pltpu.make_async_copy(xb[p], x_hbm.at[base + i, rows, :], xsem[p]),
            pltpu.make_async_copy(xsb[p], xs_hbm.at[base + i, rows, :], ssem[p]),
        )

    in_copy(0, 0, 0).start()
    in_copy(0, 1, 1).start()

    def batch_body(i, carry):
        fb = fbs[0]
        for j in range(1, B_PER_W):
            fb = jnp.where(i == j, fbs[j], fb)
        for c in range(CH_PER_B):
            p = c & 1
            ocx, ocs = out_copies(i, c, p)
            if c < 2:
                @pl.when(i > 0)
                def _():
                    ocx.wait()
                    ocs.wait()
            else:
                ocx.wait()
                ocs.wait()
            in_copy(i, c, p).wait()

            src = inb[p]
            dx = xb[p]
            dxs = xsb[p]

            def row_body(r, carry):
                for k in range(K_FULL):
                    dx[r, pl.ds(16 * k, 16)] = src[r, pl.ds(LOWER_BIN + 16 * k, 16)]
                    dxs[r, pl.ds(16 * k, 16)] = src[r, pl.ds(fb + 16 * k, 16)]
                dx[r, pl.ds(TAIL, 16)] = src[r, pl.ds(LOWER_BIN + TAIL, 16)]
                dxs[r, pl.ds(TAIL, 16)] = src[r, pl.ds(fb + TAIL, 16)]
                return carry

            lax.fori_loop(0, R_CHUNK, row_body, 0)

            ocx2, ocs2 = out_copies(i, c, p)
            ocx2.start()
            ocs2.start()

            c2 = (c + 2) % CH_PER_B
            i2 = i + (c + 2) // CH_PER_B
            if c >= 2:
                @pl.when(i2 < B_PER_W)
                def _():
                    in_copy(i2, c2, p).start()
            else:
                in_copy(i2, c2, p).start()
        return carry

    lax.fori_loop(0, B_PER_W, batch_body, 0)

    for c in (CH_PER_B - 2, CH_PER_B - 1):
        ocx, ocs = out_copies(B_PER_W - 1, c, c & 1)
        ocx.wait()
        ocs.wait()


def kernel(spec, n_shifts):
    ns32 = n_shifts.astype(jnp.int32)
    mesh = plsc.VectorSubcoreMesh(core_axis_name="c", subcore_axis_name="s")
    x, xs = pl.kernel(
        _sc_body,
        out_type=(
            jax.ShapeDtypeStruct((B, C, N_OUT), jnp.float32),
            jax.ShapeDtypeStruct((B, C, N_OUT), jnp.float32),
        ),
        mesh=mesh,
        compiler_params=pltpu.CompilerParams(use_tc_tiling_on_sc=False),
        scratch_types=(
            pltpu.VMEM((16,), jnp.int32),
            pltpu.VMEM((R_CHUNK, N_BIN), jnp.float32),
            pltpu.VMEM((R_CHUNK, N_BIN), jnp.float32),
            pltpu.VMEM((R_CHUNK, N_OUT), jnp.float32),
            pltpu.VMEM((R_CHUNK, N_OUT), jnp.float32),
            pltpu.VMEM((R_CHUNK, N_OUT), jnp.float32),
            pltpu.VMEM((R_CHUNK, N_OUT), jnp.float32),
            pltpu.SemaphoreType.DMA,
            pltpu.SemaphoreType.DMA,
            pltpu.SemaphoreType.DMA,
            pltpu.SemaphoreType.DMA,
            pltpu.SemaphoreType.DMA,
            pltpu.SemaphoreType.DMA,
        ),
    )(spec, ns32)
    return (x, xs, n_shifts)


# speedup vs baseline: 7.5094x; 4.8598x over previous
import jax
import jax.numpy as jnp
from jax import lax
from jax.experimental import pallas as pl
from jax.experimental.pallas import tpu as pltpu
from jax.experimental.pallas import tpu_sc as plsc

B, C, N_BIN = 256, 128, 360
MAX_SHIFT = 15
LOWER_BIN = 15
N_OUT = N_BIN - 2 * MAX_SHIFT

NUM_WORKERS = 32
B_PER_W = B // NUM_WORKERS
J_CHUNK = 16
N_G = 21
NBUF = 4
LAG = 2


def _chunk_j0(g):
    return N_OUT - J_CHUNK if g == N_G - 1 else J_CHUNK * g


def _sc_body(rows_hbm, ns_hbm, x_hbm, xs_hbm,
             ns2, idx0, idx1, idx2, idx3, gb0, gb1, gb2, gb3,
             gsem0, gsem1, gsem2, gsem3, osem0, osem1, osem2, osem3):
    wid = lax.axis_index("s") * 2 + lax.axis_index("c")
    base = wid * B_PER_W

    pltpu.sync_copy(ns_hbm.at[pl.ds(base, B_PER_W)], ns2.at[pl.ds(0, B_PER_W)])
    pltpu.sync_copy(ns_hbm.at[pl.ds(base, B_PER_W)], ns2.at[pl.ds(B_PER_W, B_PER_W)])
    ns_vec = ns2[...]

    lanes = lax.iota(jnp.int32, 16)
    jv = lanes >> 3
    dbv = lanes & 7
    rowbase = (base + dbv) * N_BIN + jv
    base_x = rowbase + LOWER_BIN
    base_s = rowbase + (LOWER_BIN - ns_vec)

    idxb = (idx0, idx1, idx2, idx3)
    gb = (gb0, gb1, gb2, gb3)
    gsem = (gsem0, gsem1, gsem2, gsem3)
    osem = (osem0, osem1, osem2, osem3)
    outs = (x_hbm, xs_hbm)
    bases = (base_x, base_s)

    tasks = [(win, g) for win in range(2) for g in range(N_G)]
    T = len(tasks)

    def out_copy(win, j, slot, m):
        return pltpu.make_async_copy(
            gb[slot].at[pl.ds(8 * m, 8), :],
            outs[win].at[j, pl.ds(base, B_PER_W), :],
            osem[slot],
        )

    for t in range(T + LAG):
        slot = t % NBUF
        if t < T:
            win, g = tasks[t]
            j0 = _chunk_j0(g)
            if t >= NBUF:
                pwin, pg = tasks[t - NBUF]
                pj0 = _chunk_j0(pg)
                for m in range(J_CHUNK):
                    out_copy(pwin, pj0 + m, slot, m).wait()
            bvec = bases[win] + j0
            for m in range(8):
                idxb[slot][pl.ds(16 * m, 16)] = bvec + 2 * m
            pltpu.async_copy(rows_hbm.at[idxb[slot]], gb[slot], gsem[slot])
        if t >= LAG:
            tt = t - LAG
            slot2 = tt % NBUF
            win2, g2 = tasks[tt]
            jj0 = _chunk_j0(g2)
            pltpu.make_async_copy(rows_hbm.at[idxb[slot2]], gb[slot2], gsem[slot2]).wait()
            for m in range(J_CHUNK):
                out_copy(win2, jj0 + m, slot2, m).start()

    for tt in range(max(0, T - NBUF), T):
        slot = tt % NBUF
        win, g = tasks[tt]
        j0 = _chunk_j0(g)
        for m in range(J_CHUNK):
            out_copy(win, j0 + m, slot, m).wait()


def kernel(spec, n_shifts):
    ns32 = n_shifts.astype(jnp.int32)
    rows = jnp.transpose(spec, (0, 2, 1)).reshape(B * N_BIN, C)
    mesh = plsc.VectorSubcoreMesh(core_axis_name="c", subcore_axis_name="s")
    x_t, xs_t = pl.kernel(
        _sc_body,
        out_type=(
            jax.ShapeDtypeStruct((N_OUT, B, C), jnp.float32),
            jax.ShapeDtypeStruct((N_OUT, B, C), jnp.float32),
        ),
        mesh=mesh,
        compiler_params=pltpu.CompilerParams(
            use_tc_tiling_on_sc=True, needs_layout_passes=False
        ),
        scratch_types=(
            pltpu.VMEM((16,), jnp.int32),
            pltpu.VMEM((J_CHUNK * B_PER_W,), jnp.int32),
            pltpu.VMEM((J_CHUNK * B_PER_W,), jnp.int32),
            pltpu.VMEM((J_CHUNK * B_PER_W,), jnp.int32),
            pltpu.VMEM((J_CHUNK * B_PER_W,), jnp.int32),
            pltpu.VMEM((J_CHUNK * B_PER_W, C), jnp.float32),
            pltpu.VMEM((J_CHUNK * B_PER_W, C), jnp.float32),
            pltpu.VMEM((J_CHUNK * B_PER_W, C), jnp.float32),
            pltpu.VMEM((J_CHUNK * B_PER_W, C), jnp.float32),
            pltpu.SemaphoreType.DMA,
            pltpu.SemaphoreType.DMA,
            pltpu.SemaphoreType.DMA,
            pltpu.SemaphoreType.DMA,
            pltpu.SemaphoreType.DMA,
            pltpu.SemaphoreType.DMA,
            pltpu.SemaphoreType.DMA,
            pltpu.SemaphoreType.DMA,
        ),
    )(rows, ns32)
    x = jnp.transpose(x_t, (1, 2, 0))
    xs = jnp.transpose(xs_t, (1, 2, 0))
    return (x, xs, n_shifts)


# speedup vs baseline: 7.5638x; 1.0072x over previous
import jax
import jax.numpy as jnp
from jax import lax
from jax.experimental import pallas as pl
from jax.experimental.pallas import tpu as pltpu
from jax.experimental.pallas import tpu_sc as plsc

B, C, N_BIN = 256, 128, 360
MAX_SHIFT = 15
LOWER_BIN = 15
N_OUT = N_BIN - 2 * MAX_SHIFT

NUM_WORKERS = 32
B_PER_W = B // NUM_WORKERS
J_CHUNK = 16
N_G = 21
NBUF = 6
LAG = 3


def _chunk_j0(g):
    return N_OUT - J_CHUNK if g == N_G - 1 else J_CHUNK * g


def _sc_body(rows_hbm, ns_hbm, x_hbm, xs_hbm,
             ns2, idx0, idx1, idx2, idx3, idx4, idx5, gb0, gb1, gb2, gb3, gb4, gb5,
             gsem0, gsem1, gsem2, gsem3, gsem4, gsem5,
             osem0, osem1, osem2, osem3, osem4, osem5):
    wid = lax.axis_index("s") * 2 + lax.axis_index("c")
    base = wid * B_PER_W

    pltpu.sync_copy(ns_hbm.at[pl.ds(base, B_PER_W)], ns2.at[pl.ds(0, B_PER_W)])
    pltpu.sync_copy(ns_hbm.at[pl.ds(base, B_PER_W)], ns2.at[pl.ds(B_PER_W, B_PER_W)])
    ns_vec = ns2[...]

    lanes = lax.iota(jnp.int32, 16)
    jv = lanes >> 3
    dbv = lanes & 7
    rowbase = (base + dbv) * N_BIN + jv
    base_x = rowbase + LOWER_BIN
    base_s = rowbase + (LOWER_BIN - ns_vec)

    idxb = (idx0, idx1, idx2, idx3, idx4, idx5)
    gb = (gb0, gb1, gb2, gb3, gb4, gb5)
    gsem = (gsem0, gsem1, gsem2, gsem3, gsem4, gsem5)
    osem = (osem0, osem1, osem2, osem3, osem4, osem5)
    outs = (x_hbm, xs_hbm)
    bases = (base_x, base_s)

    tasks = [(win, g) for win in range(2) for g in range(N_G)]
    T = len(tasks)

    def out_copy(win, j, slot, m):
        return pltpu.make_async_copy(
            gb[slot].at[pl.ds(8 * m, 8), :],
            outs[win].at[j, pl.ds(base, B_PER_W), :],
            osem[slot],
        )

    for t in range(T + LAG):
        slot = t % NBUF
        if t < T:
            win, g = tasks[t]
            j0 = _chunk_j0(g)
            if t >= NBUF:
                pwin, pg = tasks[t - NBUF]
                pj0 = _chunk_j0(pg)
                for m in range(J_CHUNK):
                    out_copy(pwin, pj0 + m, slot, m).wait()
            bvec = bases[win] + j0
            for m in range(8):
                idxb[slot][pl.ds(16 * m, 16)] = bvec + 2 * m
            pltpu.async_copy(rows_hbm.at[idxb[slot]], gb[slot], gsem[slot])
        if t >= LAG:
            tt = t - LAG
            slot2 = tt % NBUF
            win2, g2 = tasks[tt]
            jj0 = _chunk_j0(g2)
            pltpu.make_async_copy(rows_hbm.at[idxb[slot2]], gb[slot2], gsem[slot2]).wait()
            for m in range(J_CHUNK):
                out_copy(win2, jj0 + m, slot2, m).start()

    for tt in range(max(0, T - NBUF), T):
        slot = tt % NBUF
        win, g = tasks[tt]
        j0 = _chunk_j0(g)
        for m in range(J_CHUNK):
            out_copy(win, j0 + m, slot, m).wait()


def kernel(spec, n_shifts):
    ns32 = n_shifts.astype(jnp.int32)
    rows = jnp.transpose(spec, (0, 2, 1)).reshape(B * N_BIN, C)
    mesh = plsc.VectorSubcoreMesh(core_axis_name="c", subcore_axis_name="s")
    x_t, xs_t = pl.kernel(
        _sc_body,
        out_type=(
            jax.ShapeDtypeStruct((N_OUT, B, C), jnp.float32),
            jax.ShapeDtypeStruct((N_OUT, B, C), jnp.float32),
        ),
        mesh=mesh,
        compiler_params=pltpu.CompilerParams(
            use_tc_tiling_on_sc=True, needs_layout_passes=False
        ),
        scratch_types=(
            pltpu.VMEM((16,), jnp.int32),
            pltpu.VMEM((J_CHUNK * B_PER_W,), jnp.int32),
            pltpu.VMEM((J_CHUNK * B_PER_W,), jnp.int32),
            pltpu.VMEM((J_CHUNK * B_PER_W,), jnp.int32),
            pltpu.VMEM((J_CHUNK * B_PER_W,), jnp.int32),
            pltpu.VMEM((J_CHUNK * B_PER_W,), jnp.int32),
            pltpu.VMEM((J_CHUNK * B_PER_W,), jnp.int32),
            pltpu.VMEM((J_CHUNK * B_PER_W, C), jnp.float32),
            pltpu.VMEM((J_CHUNK * B_PER_W, C), jnp.float32),
            pltpu.VMEM((J_CHUNK * B_PER_W, C), jnp.float32),
            pltpu.VMEM((J_CHUNK * B_PER_W, C), jnp.float32),
            pltpu.VMEM((J_CHUNK * B_PER_W, C), jnp.float32),
            pltpu.VMEM((J_CHUNK * B_PER_W, C), jnp.float32),
            pltpu.SemaphoreType.DMA,
            pltpu.SemaphoreType.DMA,
            pltpu.SemaphoreType.DMA,
            pltpu.SemaphoreType.DMA,
            pltpu.SemaphoreType.DMA,
            pltpu.SemaphoreType.DMA,
            pltpu.SemaphoreType.DMA,
            pltpu.SemaphoreType.DMA,
            pltpu.SemaphoreType.DMA,
            pltpu.SemaphoreType.DMA,
            pltpu.SemaphoreType.DMA,
            pltpu.SemaphoreType.DMA,
        ),
    )(rows, ns32)
    x = jnp.transpose(x_t, (1, 2, 0))
    xs = jnp.transpose(xs_t, (1, 2, 0))
    return (x, xs, n_shifts)
